# baseline (speedup 1.0000x reference)
import functools

import jax
import jax.numpy as jnp
from jax import lax
from jax.experimental import pallas as pl
from jax.experimental.pallas import tpu as pltpu
from jax.experimental.pallas import tpu_sc as plsc

D = 128
NSLOT = 11
NC, NS = 2, 16
NW = NC * NS
R = 112
BLK = 4096


def _sc_gather_sum(table, idxT, b_pad, pairs_c0, pairs_c1):
    bps = b_pad // NS
    npairs = bps // (2 * R)
    assert pairs_c0 + pairs_c1 == npairs
    dt = table.dtype
    mesh = plsc.VectorSubcoreMesh(core_axis_name="c", subcore_axis_name="s")

    @functools.partial(
        pl.kernel,
        out_type=(jax.ShapeDtypeStruct((b_pad, D), dt),
                  jax.ShapeDtypeStruct((b_pad, D), dt),
                  jax.ShapeDtypeStruct((b_pad, D), dt)),
        mesh=mesh,
        scratch_types=[
            pltpu.VMEM((NSLOT, bps), jnp.int32),
            pltpu.VMEM((2, R, D), dt),
            pltpu.VMEM((4, R, D), dt),
            pltpu.SemaphoreType.DMA,
            pltpu.SemaphoreType.DMA,
            [pltpu.SemaphoreType.DMA] * 4,
        ],
        compiler_params=pltpu.CompilerParams(use_tc_tiling_on_sc=False),
    )
    def k(feat_hbm, idxT_hbm, self_hbm, nsa_hbm, nsb_hbm, idx_v, sbuf, nbuf,
          ssem, osem, csems):
        c = lax.axis_index("c")
        s = lax.axis_index("s")
        coff = jnp.where(c == 0, 0, pairs_c0 * 2 * R)
        my_pairs = jnp.where(c == 0, pairs_c0, pairs_c1)
        base = s * bps
        pltpu.sync_copy(idxT_hbm.at[s], idx_v)

        def chain_idx(q, off0, j):
            off = off0 + (q // 2) * R
            slot = 1 + (q % 2) * 5 + j
            return idx_v.at[slot, pl.ds(off, R)]

        def pair(pi, carry):
            off0 = coff + (2 * pi) * R
            scps = [
                pltpu.async_copy(
                    feat_hbm.at[idx_v.at[0, pl.ds(off0 + h * R, R)]],
                    sbuf.at[h], ssem)
                for h in range(2)
            ]
            prev = [
                pltpu.async_copy(
                    feat_hbm.at[chain_idx(q, off0, 0)], nbuf.at[q], csems[q])
                for q in range(4)
            ]
            for j in range(1, 5):
                nxt = []
                for q in range(4):
                    prev[q].wait()
                    nxt.append(pltpu.async_copy(
                        feat_hbm.at[chain_idx(q, off0, j)], nbuf.at[q],
                        csems[q], add=True))
                prev = nxt
            outs = []
            for h in range(2):
                scps[h].wait()
                outs.append(pltpu.async_copy(
                    sbuf.at[h], self_hbm.at[pl.ds(base + off0 + h * R, R)],
                    osem))
            for q in range(4):
                prev[q].wait()
                dst = nsa_hbm if q % 2 == 0 else nsb_hbm
                outs.append(pltpu.async_copy(
                    nbuf.at[q],
                    dst.at[pl.ds(base + off0 + (q // 2) * R, R)], osem))
            for cp in outs:
                cp.wait()
            return carry

        lax.fori_loop(0, my_pairs, pair, 0)

    return k(table, idxT)


def _tc_combine(self_rows, nsa, nsb, w1, w2, b):
    def body(x1, x2, x3, w1r, w2r, o):
        acc = jnp.dot(x1[...], w1r[...], preferred_element_type=jnp.float32)
        acc = acc + jnp.dot(x2[...] + x3[...], w2r[...],
                            preferred_element_type=jnp.float32) * \
            jnp.float32(0.1)
        o[...] = jnp.maximum(acc, 0.0)

    row_spec = pl.BlockSpec((BLK, D), lambda i: (i, 0))
    w_spec = pl.BlockSpec((D, D), lambda i: (0, 0))
    return pl.pallas_call(
        body,
        grid=((b + BLK - 1) // BLK,),
        in_specs=[row_spec] * 3 + [w_spec] * 2,
        out_specs=row_spec,
        out_shape=jax.ShapeDtypeStruct((b, D), jnp.float32),
    )(self_rows, nsa, nsb, w1, w2)


def kernel(features, weight, nodes, neigh_idx):
    b = nodes.shape[0]
    step = NW * R * 2
    b_pad = ((b + step - 1) // step) * step

    idx_all = jnp.concatenate(
        [nodes[:, None].astype(jnp.int32), neigh_idx.astype(jnp.int32)],
        axis=1).T
    idxT = jnp.pad(idx_all, ((0, 0), (0, b_pad - b)))
    idxT = idxT.reshape(NSLOT, NS, b_pad // NS).transpose(1, 0, 2)

    npairs = b_pad // (NS * 2 * R)
    pairs_c0 = (9 * npairs) // 14
    self_rows, nsa, nsb = _sc_gather_sum(features, idxT, b_pad, pairs_c0,
                                         npairs - pairs_c0)
    return _tc_combine(self_rows, nsa, nsb, weight[:D], weight[D:], b)


# speedup vs baseline: 1.0148x; 1.0148x over previous
import functools

import jax
import jax.numpy as jnp
from jax import lax
from jax.experimental import pallas as pl
from jax.experimental.pallas import tpu as pltpu
from jax.experimental.pallas import tpu_sc as plsc

D = 128
NSLOT = 11
NC, NS = 2, 16
NW = NC * NS
R = 112
BLK = 4096


def _sc_gather_sum(table, idxT, b_pad, pairs_c0, pairs_c1):
    bps = b_pad // NS
    npairs = bps // (2 * R)
    assert pairs_c0 + pairs_c1 == npairs
    dt = table.dtype
    mesh = plsc.VectorSubcoreMesh(core_axis_name="c", subcore_axis_name="s")

    @functools.partial(
        pl.kernel,
        out_type=(jax.ShapeDtypeStruct((b_pad, D), dt),
                  jax.ShapeDtypeStruct((b_pad, D), dt),
                  jax.ShapeDtypeStruct((b_pad, D), dt)),
        mesh=mesh,
        scratch_types=[
            pltpu.VMEM((NSLOT, bps), jnp.int32),
            pltpu.VMEM((2, R, D), dt),
            pltpu.VMEM((4, R, D), dt),
            pltpu.SemaphoreType.DMA,
            pltpu.SemaphoreType.DMA,
            [pltpu.SemaphoreType.DMA] * 4,
        ],
        compiler_params=pltpu.CompilerParams(use_tc_tiling_on_sc=False),
    )
    def k(feat_hbm, idxT_hbm, self_hbm, nsa_hbm, nsb_hbm, idx_v, sbuf, nbuf,
          ssem, osem, csems):
        c = lax.axis_index("c")
        s = lax.axis_index("s")
        coff = jnp.where(c == 0, 0, pairs_c0 * 2 * R)
        my_pairs = jnp.where(c == 0, pairs_c0, pairs_c1)
        base = s * bps
        pltpu.sync_copy(idxT_hbm.at[s], idx_v)

        def chain_idx(q, off0, j):
            off = off0 + (q // 2) * R
            slot = 1 + (q % 2) * 5 + j
            return idx_v.at[slot, pl.ds(off, R)]

        def pair(pi, carry):
            off0 = coff + (2 * pi) * R
            scps = [
                pltpu.async_copy(
                    feat_hbm.at[idx_v.at[0, pl.ds(off0 + h * R, R)]],
                    sbuf.at[h], ssem)
                for h in range(2)
            ]
            prev = [
                pltpu.async_copy(
                    feat_hbm.at[chain_idx(q, off0, 0)], nbuf.at[q], csems[q])
                for q in range(4)
            ]
            for j in range(1, 5):
                nxt = []
                for q in range(4):
                    prev[q].wait()
                    nxt.append(pltpu.async_copy(
                        feat_hbm.at[chain_idx(q, off0, j)], nbuf.at[q],
                        csems[q], add=True))
                prev = nxt
            outs = []
            for h in range(2):
                scps[h].wait()
                outs.append(pltpu.async_copy(
                    sbuf.at[h], self_hbm.at[pl.ds(base + off0 + h * R, R)],
                    osem))
            for q in range(4):
                prev[q].wait()
                dst = nsa_hbm if q % 2 == 0 else nsb_hbm
                outs.append(pltpu.async_copy(
                    nbuf.at[q],
                    dst.at[pl.ds(base + off0 + (q // 2) * R, R)], osem))
            for cp in outs:
                cp.wait()
            return carry

        lax.fori_loop(0, my_pairs, pair, 0)

    return k(table, idxT)


def _tc_combine(self_rows, nsa, nsb, w1, w2, b):
    def body(x1, x2, x3, w1r, w2r, o):
        acc = jnp.dot(x1[...], w1r[...], preferred_element_type=jnp.float32)
        acc = acc + jnp.dot(x2[...] + x3[...], w2r[...],
                            preferred_element_type=jnp.float32) * \
            jnp.float32(0.1)
        o[...] = jnp.maximum(acc, 0.0)

    row_spec = pl.BlockSpec((BLK, D), lambda i: (i, 0))
    w_spec = pl.BlockSpec((D, D), lambda i: (0, 0))
    return pl.pallas_call(
        body,
        grid=((b + BLK - 1) // BLK,),
        in_specs=[row_spec] * 3 + [w_spec] * 2,
        out_specs=row_spec,
        out_shape=jax.ShapeDtypeStruct((b, D), jnp.float32),
    )(self_rows, nsa, nsb, w1, w2)


def kernel(features, weight, nodes, neigh_idx):
    b = nodes.shape[0]
    step = NW * R * 2
    b_pad = ((b + step - 1) // step) * step

    idx_all = jnp.concatenate(
        [nodes[:, None].astype(jnp.int32), neigh_idx.astype(jnp.int32)],
        axis=1).T
    idxT = jnp.pad(idx_all, ((0, 0), (0, b_pad - b)))
    idxT = idxT.reshape(NSLOT, NS, b_pad // NS).transpose(1, 0, 2)

    npairs = b_pad // (NS * 2 * R)
    pairs_c0 = npairs // 2
    self_rows, nsa, nsb = _sc_gather_sum(features, idxT, b_pad, pairs_c0,
                                         npairs - pairs_c0)
    return _tc_combine(self_rows, nsa, nsb, weight[:D], weight[D:], b)
